# Initial kernel scaffold; baseline (speedup 1.0000x reference)
import jax
import jax.numpy as jnp
from jax.experimental import pallas as pl


def kernel(x, edge_index, edge_attr, atom_emb, W, att_src, att_dst, att_edge, bias, edge_emb, bn_gamma, bn_beta, pred_W, pred_b):
    raise NotImplementedError("write your pallas kernel here")


# speedup vs baseline: 26.9183x; 26.9183x over previous
import functools

import jax
import jax.numpy as jnp
from jax import lax
from jax.experimental import pallas as pl
from jax.experimental.pallas import tpu as pltpu
from jax.experimental.pallas import tpu_sc as plsc

L = 3
D = 128
H = 8
C = 16
N = 10000
E = 320000

NC = 2
NS = 16
NW = NC * NS

NP = 10240
B = 32
NB = 316
EW = NB * B
EP = NW * EW
RPW = NP // NS
RPW8 = NP // 8 // NS

_BN_SCALE = 1.0 / (1.0 + 1e-5) ** 0.5


_RB = 512
_GRID = NP // _RB


def _embed_body(x_ref, emb_ref, o_ref):
  xv = x_ref[0, 0, :]
  oh = (xv[:, None] == lax.broadcasted_iota(jnp.int32, (1, 16), 1)
        ).astype(jnp.float32)
  o_ref[...] = jnp.dot(oh, emb_ref[...],
                       preferred_element_type=jnp.float32)


def _tc_embed(xp, atom_emb):
  xp3 = xp.reshape(_GRID, 1, _RB)
  return pl.pallas_call(
      _embed_body,
      grid=(_GRID,),
      in_specs=[
          pl.BlockSpec((1, 1, _RB), lambda i: (i, 0, 0)),
          pl.BlockSpec((16, D), lambda i: (0, 0)),
      ],
      out_specs=pl.BlockSpec((_RB, D), lambda i: (i, 0)),
      out_shape=jax.ShapeDtypeStruct((NP, D), jnp.float32),
  )(xp3, atom_emb)


def _pre_body(h_ref, w_ref, a2_ref, xs_ref, asd8_ref):
  xs = jnp.dot(h_ref[...], w_ref[...], preferred_element_type=jnp.float32)
  asd = jnp.dot(xs, a2_ref[...], preferred_element_type=jnp.float32)
  xs_ref[...] = xs
  asd8_ref[...] = asd


def _tc_pre(hp, w_l, a2_l):
  return pl.pallas_call(
      _pre_body,
      grid=(_GRID,),
      in_specs=[
          pl.BlockSpec((_RB, D), lambda i: (i, 0)),
          pl.BlockSpec((D, D), lambda i: (0, 0)),
          pl.BlockSpec((D, 16), lambda i: (0, 0)),
      ],
      out_specs=[
          pl.BlockSpec((_RB, D), lambda i: (i, 0)),
          pl.BlockSpec((_RB, 16), lambda i: (i, 0)),
      ],
      out_shape=[
          jax.ShapeDtypeStruct((NP, D), jnp.float32),
          jax.ShapeDtypeStruct((NP, 16), jnp.float32),
      ],
  )(hp, w_l, a2_l)


def _post_body(acc0_ref, acc1_ref, aux0_ref, aux1_ref, xs_ref, asd_ref,
               h_ref, ea_ref, aet_ref, rex_ref, bias_ref, gam_ref, bet_ref,
               o_ref):
  acc = acc0_ref[...] + acc1_ref[...]
  aux = aux0_ref[...] + aux1_ref[...]
  den_e = aux[:, :8]
  cc = aux[:, 8:12]
  cnt = jnp.sum(cc, axis=1)
  inv = 1.0 / jnp.maximum(cnt, 1.0)
  loop_attr = jnp.dot(cc, ea_ref[...],
                      preferred_element_type=jnp.float32) * inv[:, None]
  ael = jnp.dot(cc, aet_ref[...],
                preferred_element_type=jnp.float32)[:, :8] * inv[:, None]
  asd = asd_ref[...]
  a_src = asd[:, :8]
  a_dst = asd[:, 8:]
  al = a_dst + a_src + ael
  al = jnp.where(al > 0, al, 0.2 * al)
  exl = jnp.exp(al)
  rex = rex_ref[...]
  exl_e = jnp.dot(exl, rex, preferred_element_type=jnp.float32)
  den = jnp.dot(den_e + exl, rex, preferred_element_type=jnp.float32)
  acc_tot = acc + exl_e * (xs_ref[...] + loop_attr)
  out = acc_tot / den + bias_ref[...]
  out = out * (gam_ref[...] * _BN_SCALE) + bet_ref[...]
  o_ref[...] = jnp.maximum(out, 0.0) + h_ref[...]


def _tc_post(acc0, acc1, aux0, aux1, xs, asd, hp, ea_l, aet16_l, rex,
             bias_l, gam_l, bet_l):
  blk = lambda w: pl.BlockSpec((_RB, w), lambda i: (i, 0))
  full = lambda a, b: pl.BlockSpec((a, b), lambda i: (0, 0))
  return pl.pallas_call(
      _post_body,
      grid=(_GRID,),
      in_specs=[
          blk(D), blk(D), blk(16), blk(16), blk(D), blk(16), blk(D),
          full(4, D), full(4, 16), full(8, D),
          full(1, D), full(1, D), full(1, D),
      ],
      out_specs=blk(D),
      out_shape=jax.ShapeDtypeStruct((NP, D), jnp.float32),
  )(acc0, acc1, aux0, aux1, xs, asd, hp, ea_l, aet16_l, rex,
    bias_l, gam_l, bet_l)


def _uv_body(h_ref, pw_ref, pb_ref, o_ref):
  o_ref[...] = jnp.dot(h_ref[...], pw_ref[...],
                       preferred_element_type=jnp.float32) + pb_ref[...]


def _tc_uv(hp, pw16, pb16):
  return pl.pallas_call(
      _uv_body,
      grid=(_GRID,),
      in_specs=[
          pl.BlockSpec((_RB, D), lambda i: (i, 0)),
          pl.BlockSpec((D, 16), lambda i: (0, 0)),
          pl.BlockSpec((1, 16), lambda i: (0, 0)),
      ],
      out_specs=pl.BlockSpec((_RB, 16), lambda i: (i, 0)),
      out_shape=jax.ShapeDtypeStruct((NP, 16), jnp.float32),
  )(hp, pw16, pb16)


def _edge_body(src_hbm, dst_hbm, attr_hbm, xs_hbm, asd8_hbm, ea_hbm,
               aet_hbm, accm_out, accaux_out,
               accm_sh, accaux_sh,
               srcv, dstv, attrv, srcv8, dstv8, xbuf, sbuf, dbuf, auxb,
               eav, aetv):
  c = lax.axis_index("c")
  s = lax.axis_index("s")
  wid = c * NS + s
  lanes = lax.iota(jnp.int32, 16)
  zero16 = jnp.zeros((16,), jnp.float32)

  def _zrow(i, _):
    for k in range(8):
      auxb[i, pl.ds(k * 16, 16)] = zero16
    return _
  lax.fori_loop(0, B, _zrow, None)

  def _zcp(j, _):
    pltpu.sync_copy(auxb, accm_sh.at[pl.ds(s * RPW + j * B, B)])
    return _
  lax.fori_loop(0, RPW // B, _zcp, None)

  def _zcpa(j, _):
    pltpu.sync_copy(auxb.at[pl.ds(0, 16)],
                    accaux_sh.at[pl.ds(s * RPW8 + j * 16, 16)])
    return _
  lax.fori_loop(0, RPW8 // 16, _zcpa, None)

  pltpu.sync_copy((ea_hbm, aet_hbm), (eav, aetv))
  plsc.subcore_barrier()

  def _block(blk, _):
    base = wid * EW + blk * B
    pltpu.sync_copy(
        (src_hbm.at[pl.ds(base, B)], dst_hbm.at[pl.ds(base, B)],
         attr_hbm.at[pl.ds(base, B)]),
        (srcv, dstv, attrv))

    def _mkidx(g, _):
      srcv8[pl.ds(g * 16, 16)] = lax.shift_right_logical(
          srcv[pl.ds(g * 16, 16)], 3)
      dstv8[pl.ds(g * 16, 16)] = lax.shift_right_logical(
          dstv[pl.ds(g * 16, 16)], 3)
      return _
    lax.fori_loop(0, B // 16, _mkidx, None)

    pltpu.sync_copy(
        (xs_hbm.at[srcv], asd8_hbm.at[srcv8], asd8_hbm.at[dstv8]),
        (xbuf, sbuf, dbuf.at[pl.ds(0, B)]))

    def _group(g, _):
      srcg = srcv[pl.ds(g * 16, 16)]
      dstg = dstv[pl.ds(g * 16, 16)]
      attrg = attrv[pl.ds(g * 16, 16)]
      for j in range(16):
        e = g * 16 + j
        src_j = srcg[j]
        dst_j = dstg[j]
        a_e = attrg[j]
        maskf = jnp.where(src_j != dst_j, 1.0, 0.0)
        so8 = (src_j & 7) * 16
        do8 = (dst_j & 7) * 16
        sv = (sbuf[e, pl.ds(so8, 16)] + dbuf[e, pl.ds(do8 + 8, 16)]
              + aetv[a_e, pl.ds(0, 16)])
        tv = jnp.where(sv > 0, sv, 0.2 * sv)
        exv = jnp.exp(tv) * maskf
        oh = jnp.where(lanes == 8 + a_e, maskf, 0.0)
        aux16 = jnp.where(lanes < 8, exv, oh)
        auxb[e, pl.ds(do8, 16)] = aux16
        for h in range(H):
          bex = jnp.full((16,), exv[h])
          xv = xbuf[e, pl.ds(h * 16, 16)]
          ev = eav[a_e, pl.ds(h * 16, 16)]
          xbuf[e, pl.ds(h * 16, 16)] = bex * (xv + ev)
      return _
    lax.fori_loop(0, B // 16, _group, None)

    pltpu.sync_copy(xbuf, accm_sh.at[dstv], add=True)
    pltpu.sync_copy(auxb, accaux_sh.at[dstv8], add=True)

    def _zslot(g, _):
      dstg = dstv[pl.ds(g * 16, 16)]
      for j in range(16):
        auxb[g * 16 + j, pl.ds((dstg[j] & 7) * 16, 16)] = zero16
      return _
    lax.fori_loop(0, B // 16, _zslot, None)
    return _
  lax.fori_loop(0, NB, _block, None)

  plsc.subcore_barrier()

  def _cpo(j, _):
    r = s * RPW + j * B
    pltpu.sync_copy(accm_sh.at[pl.ds(r, B)], xbuf)
    pltpu.sync_copy(xbuf, accm_out.at[c, pl.ds(r, B)])
    return _
  lax.fori_loop(0, RPW // B, _cpo, None)

  def _cpoa(j, _):
    r = s * RPW8 + j * 16
    pltpu.sync_copy(accaux_sh.at[pl.ds(r, 16)], sbuf.at[pl.ds(0, 16)])
    pltpu.sync_copy(sbuf.at[pl.ds(0, 16)], accaux_out.at[c, pl.ds(r, 16)])
    return _
  lax.fori_loop(0, RPW8 // 16, _cpoa, None)


_edge_kernel = pl.kernel(
    _edge_body,
    out_type=[
        jax.ShapeDtypeStruct((NC, NP, D), jnp.float32),
        jax.ShapeDtypeStruct((NC, NP // 8, 128), jnp.float32),
    ],
    mesh=plsc.VectorSubcoreMesh(
        core_axis_name="c", subcore_axis_name="s",
        num_cores=NC, num_subcores=NS),
    scratch_types=[
        pltpu.VMEM_SHARED((NP, D), jnp.float32),
        pltpu.VMEM_SHARED((NP // 8, 128), jnp.float32),
        pltpu.VMEM((B,), jnp.int32),
        pltpu.VMEM((B,), jnp.int32),
        pltpu.VMEM((B,), jnp.int32),
        pltpu.VMEM((B,), jnp.int32),
        pltpu.VMEM((B,), jnp.int32),
        pltpu.VMEM((B, 128), jnp.float32),
        pltpu.VMEM((B, 128), jnp.float32),
        pltpu.VMEM((B + 8, 128), jnp.float32),
        pltpu.VMEM((B, 128), jnp.float32),
        pltpu.VMEM((4, D), jnp.float32),
        pltpu.VMEM((4, 16), jnp.float32),
    ],
)


EWP = E // NW


def _pred_body(row_hbm, col_hbm, u_hbm, v_hbm, out_hbm,
               rowv, colv, uv, vv, ob):
  c = lax.axis_index("c")
  s = lax.axis_index("s")
  wid = c * NS + s
  base = wid * EWP
  pltpu.sync_copy(
      (row_hbm.at[pl.ds(base, EWP)], col_hbm.at[pl.ds(base, EWP)],
       u_hbm, v_hbm),
      (rowv, colv, uv, vv))

  lanes = lax.iota(jnp.int32, 16)
  zero16 = jnp.zeros((16,), jnp.float32)

  def _go(i, _):
    r16 = rowv[pl.ds(i * 16, 16)]
    c16 = colv[pl.ds(i * 16, 16)]
    ov = zero16
    for j in range(16):
      uvec = uv[pl.ds(r16[j], 16)]
      vvec = vv[pl.ds(c16[j], 16)]
      ov = jnp.where(lanes == j, uvec[0] + vvec[0], ov)
    ob[pl.ds(i * 16, 16)] = ov
    return _
  lax.fori_loop(0, EWP // 16, _go, None)
  pltpu.sync_copy(ob, out_hbm.at[pl.ds(base, EWP)])


_pred_kernel = pl.kernel(
    _pred_body,
    out_type=jax.ShapeDtypeStruct((E,), jnp.float32),
    mesh=plsc.VectorSubcoreMesh(
        core_axis_name="c", subcore_axis_name="s",
        num_cores=NC, num_subcores=NS),
    scratch_types=[
        pltpu.VMEM((EWP,), jnp.int32),
        pltpu.VMEM((EWP,), jnp.int32),
        pltpu.VMEM((NP,), jnp.float32),
        pltpu.VMEM((NP,), jnp.float32),
        pltpu.VMEM((EWP,), jnp.float32),
    ],
)


def _blockdiag(att):
  a = att.reshape(H, C)
  return (a[:, :, None] * jnp.eye(H, dtype=jnp.float32)[:, None, :]
          ).reshape(D, H)


def kernel(x, edge_index, edge_attr, atom_emb, W, att_src, att_dst, att_edge,
           bias, edge_emb, bn_gamma, bn_beta, pred_W, pred_b):
  f32 = jnp.float32
  src = edge_index[0].astype(jnp.int32)
  dst = edge_index[1].astype(jnp.int32)
  attr = edge_attr.astype(jnp.int32)

  padE = EP - E
  zpad = jnp.zeros((padE,), jnp.int32)
  srcp = jnp.concatenate([src, zpad])
  dstp = jnp.concatenate([dst, zpad])
  attrp = jnp.concatenate([attr, zpad])

  xp = jnp.concatenate([x.astype(jnp.int32),
                        jnp.zeros((NP - N,), jnp.int32)])

  rex = jnp.kron(jnp.eye(H, dtype=f32), jnp.ones((1, C), f32))

  hp = _tc_embed(xp, atom_emb.astype(f32))

  for l in range(L):
    a2 = jnp.concatenate(
        [_blockdiag(att_src[l]), _blockdiag(att_dst[l])], axis=1)
    ea_l = edge_emb[l]
    aet = (ea_l.reshape(4, H, C) * att_edge[l].reshape(1, H, C)).sum(-1)
    aet16 = jnp.concatenate([aet, jnp.zeros((4, 8), f32)], axis=1)

    xs, asd = _tc_pre(hp, W[l], a2)
    asd8 = asd.reshape(NP // 8, 128)
    accm, accaux = _edge_kernel(srcp, dstp, attrp, xs, asd8, ea_l, aet16)
    aux = accaux.reshape(NC, NP, 16)
    hp = _tc_post(accm[0], accm[1], aux[0], aux[1], xs, asd, hp,
                  ea_l, aet16, rex,
                  bias[l].reshape(1, D), bn_gamma[l].reshape(1, D),
                  bn_beta[l].reshape(1, D))

  pw16 = jnp.concatenate(
      [pred_W[:D], pred_W[D:], jnp.zeros((D, 14), f32)], axis=1)
  pb16 = jnp.zeros((1, 16), f32).at[0, 0].set(pred_b[0])
  uvt = _tc_uv(hp, pw16, pb16)
  u = uvt[:, 0]
  v = uvt[:, 1]

  pred = _pred_kernel(src, dst, u, v)
  return pred.reshape(E, 1)


# speedup vs baseline: 43.5235x; 1.6169x over previous
import functools

import jax
import jax.numpy as jnp
from jax import lax
from jax.experimental import pallas as pl
from jax.experimental.pallas import tpu as pltpu
from jax.experimental.pallas import tpu_sc as plsc

L = 3
D = 128
H = 8
C = 16
N = 10000
E = 320000

NC = 2
NS = 16
NW = NC * NS

NP = 10240
B = 32
NB = 316
EW = NB * B
EP = NW * EW
RPW = NP // NS
RPW8 = NP // 8 // NS

_BN_SCALE = 1.0 / (1.0 + 1e-5) ** 0.5


_RB = 512
_GRID = NP // _RB


def _embed_body(x_ref, emb_ref, o_ref):
  xv = x_ref[0, 0, :]
  oh = (xv[:, None] == lax.broadcasted_iota(jnp.int32, (1, 16), 1)
        ).astype(jnp.float32)
  o_ref[...] = jnp.dot(oh, emb_ref[...],
                       preferred_element_type=jnp.float32)


def _tc_embed(xp, atom_emb):
  xp3 = xp.reshape(_GRID, 1, _RB)
  return pl.pallas_call(
      _embed_body,
      grid=(_GRID,),
      in_specs=[
          pl.BlockSpec((1, 1, _RB), lambda i: (i, 0, 0)),
          pl.BlockSpec((16, D), lambda i: (0, 0)),
      ],
      out_specs=pl.BlockSpec((_RB, D), lambda i: (i, 0)),
      out_shape=jax.ShapeDtypeStruct((NP, D), jnp.float32),
  )(xp3, atom_emb)


def _pre_body(h_ref, w_ref, a2_ref, xs_ref, asd_ref, ads_ref):
  xs = jnp.dot(h_ref[...], w_ref[...], preferred_element_type=jnp.float32)
  asd = jnp.dot(xs, a2_ref[...], preferred_element_type=jnp.float32)
  xs_ref[...] = xs
  asd_ref[...] = asd
  ads_ref[...] = jnp.concatenate([asd[:, 8:], asd[:, :8]], axis=1)


def _tc_pre(hp, w_l, a2_l):
  return pl.pallas_call(
      _pre_body,
      grid=(_GRID,),
      in_specs=[
          pl.BlockSpec((_RB, D), lambda i: (i, 0)),
          pl.BlockSpec((D, D), lambda i: (0, 0)),
          pl.BlockSpec((D, 16), lambda i: (0, 0)),
      ],
      out_specs=[
          pl.BlockSpec((_RB, D), lambda i: (i, 0)),
          pl.BlockSpec((_RB, 16), lambda i: (i, 0)),
          pl.BlockSpec((_RB, 16), lambda i: (i, 0)),
      ],
      out_shape=[
          jax.ShapeDtypeStruct((NP, D), jnp.float32),
          jax.ShapeDtypeStruct((NP, 16), jnp.float32),
          jax.ShapeDtypeStruct((NP, 16), jnp.float32),
      ],
  )(hp, w_l, a2_l)


def _post_body(acc0_ref, acc1_ref, aux0_ref, aux1_ref, xs_ref, asd_ref,
               h_ref, ea_ref, aet_ref, rex_ref, bias_ref, gam_ref, bet_ref,
               o_ref):
  acc = acc0_ref[...] + acc1_ref[...]
  aux = aux0_ref[...] + aux1_ref[...]
  den_e = aux[:, :8]
  cc = aux[:, 8:12]
  cnt = jnp.sum(cc, axis=1)
  inv = 1.0 / jnp.maximum(cnt, 1.0)
  loop_attr = jnp.dot(cc, ea_ref[...],
                      preferred_element_type=jnp.float32) * inv[:, None]
  ael = jnp.dot(cc, aet_ref[...],
                preferred_element_type=jnp.float32)[:, :8] * inv[:, None]
  asd = asd_ref[...]
  a_src = asd[:, :8]
  a_dst = asd[:, 8:]
  al = a_dst + a_src + ael
  al = jnp.where(al > 0, al, 0.2 * al)
  exl = jnp.exp(al)
  rex = rex_ref[...]
  exl_e = jnp.dot(exl, rex, preferred_element_type=jnp.float32)
  den = jnp.dot(den_e + exl, rex, preferred_element_type=jnp.float32)
  acc_tot = acc + exl_e * (xs_ref[...] + loop_attr)
  out = acc_tot / den + bias_ref[...]
  out = out * (gam_ref[...] * _BN_SCALE) + bet_ref[...]
  o_ref[...] = jnp.maximum(out, 0.0) + h_ref[...]


def _tc_post(acc0, acc1, aux0, aux1, xs, asd, hp, ea_l, aet16_l, rex,
             bias_l, gam_l, bet_l):
  blk = lambda w: pl.BlockSpec((_RB, w), lambda i: (i, 0))
  full = lambda a, b: pl.BlockSpec((a, b), lambda i: (0, 0))
  return pl.pallas_call(
      _post_body,
      grid=(_GRID,),
      in_specs=[
          blk(D), blk(D), blk(16), blk(16), blk(D), blk(16), blk(D),
          full(4, D), full(4, 16), full(8, D),
          full(1, D), full(1, D), full(1, D),
      ],
      out_specs=blk(D),
      out_shape=jax.ShapeDtypeStruct((NP, D), jnp.float32),
  )(acc0, acc1, aux0, aux1, xs, asd, hp, ea_l, aet16_l, rex,
    bias_l, gam_l, bet_l)


def _uv_body(h_ref, pw_ref, pb_ref, o_ref):
  o_ref[...] = jnp.dot(h_ref[...], pw_ref[...],
                       preferred_element_type=jnp.float32) + pb_ref[...]


def _tc_uv(hp, pw16, pb16):
  return pl.pallas_call(
      _uv_body,
      grid=(_GRID,),
      in_specs=[
          pl.BlockSpec((_RB, D), lambda i: (i, 0)),
          pl.BlockSpec((D, 16), lambda i: (0, 0)),
          pl.BlockSpec((1, 16), lambda i: (0, 0)),
      ],
      out_specs=pl.BlockSpec((_RB, 16), lambda i: (i, 0)),
      out_shape=jax.ShapeDtypeStruct((NP, 16), jnp.float32),
  )(hp, pw16, pb16)


NBX = NW * NB + 4


def _edge_body(epk_hbm, xs_hbm, asd8_hbm, ads8_hbm,
               ea_hbm, aet_hbm, accm_out, accaux_out,
               accm_sh, accaux_sh,
               srcA, dstA, attrA, src8A, dst8A,
               srcB, dstB, attrB, src8B, dst8B,
               xbufA, sbufA, dbufA, epkA,
               xbufB, sbufB, dbufB, epkB,
               auxb, eav, aetv,
               gxA, gsA, gdA, giA, gxB, gsB, gdB, giB):
  c = lax.axis_index("c")
  s = lax.axis_index("s")
  wid = c * NS + s
  lanes = lax.iota(jnp.int32, 16)
  zero16 = jnp.zeros((16,), jnp.float32)
  bbase = wid * NB

  A = (srcA, dstA, attrA, src8A, dst8A, xbufA, sbufA, dbufA, epkA,
       gxA, gsA, gdA, giA)
  Bb = (srcB, dstB, attrB, src8B, dst8B, xbufB, sbufB, dbufB, epkB,
        gxB, gsB, gdB, giB)

  def _zrow(i, _):
    for k in range(8):
      xbufA[i, pl.ds(k * 16, 16)] = zero16
      auxb[i, pl.ds(k * 16, 16)] = zero16
    return _
  lax.fori_loop(0, B, _zrow, None)

  def _zcp(j, _):
    pltpu.sync_copy(xbufA, accm_sh.at[pl.ds(s * RPW + j * B, B)])
    return _
  lax.fori_loop(0, RPW // B, _zcp, None)

  def _zcpa(j, _):
    pltpu.sync_copy(auxb.at[pl.ds(0, 16)],
                    accaux_sh.at[pl.ds(s * RPW8 + j * 16, 16)])
    return _
  lax.fori_loop(0, RPW8 // 16, _zcpa, None)

  pltpu.sync_copy((ea_hbm, aet_hbm), (eav, aetv))
  plsc.subcore_barrier()

  def i_desc(blk, bufs):
    epk, gi = bufs[8], bufs[12]
    return pltpu.make_async_copy(epk_hbm.at[pl.ds(bbase + blk, 1)], epk, gi)

  def prep(bufs):
    srcv, dstv, attrv, srcv8, dstv8, epk = bufs[:5] + (bufs[8],)
    for g in range(B // 16):
      o = g * 16
      sg = epk[0, 0, pl.ds(o, 16)]
      dg = epk[0, 1, pl.ds(o, 16)]
      srcv[pl.ds(o, 16)] = sg
      dstv[pl.ds(o, 16)] = dg
      attrv[pl.ds(o, 16)] = epk[0, 2, pl.ds(o, 16)]
      srcv8[pl.ds(o, 16)] = lax.shift_right_logical(sg, 3)
      dstv8[pl.ds(o, 16)] = lax.shift_right_logical(dg, 3)

  def g_descs(bufs):
    srcv8, dstv8 = bufs[3], bufs[4]
    srcv, xbuf, sbuf, dbuf = bufs[0], bufs[5], bufs[6], bufs[7]
    gx, gs, gd = bufs[9:12]
    return (pltpu.make_async_copy(xs_hbm.at[srcv], xbuf, gx),
            pltpu.make_async_copy(asd8_hbm.at[srcv8], sbuf, gs),
            pltpu.make_async_copy(ads8_hbm.at[dstv8], dbuf, gd))

  def g_start(bufs):
    for d in g_descs(bufs):
      d.start()

  def g_wait(bufs):
    for d in g_descs(bufs):
      d.wait()

  def compute(bufs):
    srcv, dstv, attrv = bufs[:3]
    xbuf, sbuf, dbuf = bufs[5:8]

    def _group(g, _):
      srcg = srcv[pl.ds(g * 16, 16)]
      dstg = dstv[pl.ds(g * 16, 16)]
      attrg = attrv[pl.ds(g * 16, 16)]
      for j in range(16):
        e = g * 16 + j
        src_j = srcg[j]
        dst_j = dstg[j]
        a_e = attrg[j]
        maskf = jnp.where(src_j != dst_j, 1.0, 0.0)
        so8 = (src_j & 7) * 16
        do8 = (dst_j & 7) * 16
        sv = (sbuf[e, pl.ds(so8, 16)] + dbuf[e, pl.ds(do8, 16)]
              + aetv[a_e, pl.ds(0, 16)])
        tv = jnp.where(sv > 0, sv, 0.2 * sv)
        exv = jnp.exp(tv) * maskf
        oh = jnp.where(lanes == 8 + a_e, maskf, 0.0)
        aux16 = jnp.where(lanes < 8, exv, oh)
        auxb[e, pl.ds(do8, 16)] = aux16
        for h in range(H):
          bex = jnp.full((16,), exv[h])
          xv = xbuf[e, pl.ds(h * 16, 16)]
          ev = eav[a_e, pl.ds(h * 16, 16)]
          xbuf[e, pl.ds(h * 16, 16)] = bex * (xv + ev)
      return _
    lax.fori_loop(0, B // 16, _group, None)

  def scatter_rezero(bufs):
    dstv, dstv8, xbuf = bufs[1], bufs[4], bufs[5]
    pltpu.sync_copy(xbuf, accm_sh.at[dstv], add=True)
    pltpu.sync_copy(auxb, accaux_sh.at[dstv8], add=True)

    def _z(g, _):
      dstg = dstv[pl.ds(g * 16, 16)]
      for j in range(16):
        auxb[g * 16 + j, pl.ds((dstg[j] & 7) * 16, 16)] = zero16
      return _
    lax.fori_loop(0, B // 16, _z, None)

  i_desc(0, A).start()
  i_desc(1, Bb).start()
  i_desc(0, A).wait()
  prep(A)
  g_start(A)
  i_desc(1, Bb).wait()
  prep(Bb)
  g_start(Bb)
  i_desc(2, A).start()
  i_desc(3, Bb).start()

  def _step(blk, bufs):
    g_wait(bufs)
    compute(bufs)
    scatter_rezero(bufs)
    i_desc(blk + 2, bufs).wait()
    prep(bufs)
    g_start(bufs)
    i_desc(blk + 4, bufs).start()

  def _iter(i, _):
    _step(2 * i, A)
    _step(2 * i + 1, Bb)
    return _
  lax.fori_loop(0, NB // 2, _iter, None)

  g_wait(A)
  g_wait(Bb)
  i_desc(NB + 2, A).wait()
  i_desc(NB + 3, Bb).wait()

  plsc.subcore_barrier()

  def _cpo(j, _):
    r = s * RPW + j * B
    pltpu.sync_copy(accm_sh.at[pl.ds(r, B)], xbufA)
    pltpu.sync_copy(xbufA, accm_out.at[c, pl.ds(r, B)])
    return _
  lax.fori_loop(0, RPW // B, _cpo, None)

  def _cpoa(j, _):
    r = s * RPW8 + j * 16
    pltpu.sync_copy(accaux_sh.at[pl.ds(r, 16)], dbufA.at[pl.ds(0, 16)])
    pltpu.sync_copy(dbufA.at[pl.ds(0, 16)], accaux_out.at[c, pl.ds(r, 16)])
    return _
  lax.fori_loop(0, RPW8 // 16, _cpoa, None)


def _i32buf():
  return pltpu.VMEM((B,), jnp.int32)


_edge_kernel = pl.kernel(
    _edge_body,
    out_type=[
        jax.ShapeDtypeStruct((NC, NP, D), jnp.float32),
        jax.ShapeDtypeStruct((NC, NP // 8, 128), jnp.float32),
    ],
    mesh=plsc.VectorSubcoreMesh(
        core_axis_name="c", subcore_axis_name="s",
        num_cores=NC, num_subcores=NS),
    scratch_types=[
        pltpu.VMEM_SHARED((NP, D), jnp.float32),
        pltpu.VMEM_SHARED((NP // 8, 128), jnp.float32),
        _i32buf(), _i32buf(), _i32buf(), _i32buf(), _i32buf(),
        _i32buf(), _i32buf(), _i32buf(), _i32buf(), _i32buf(),
        pltpu.VMEM((B, 128), jnp.float32),
        pltpu.VMEM((B, 128), jnp.float32),
        pltpu.VMEM((B, 128), jnp.float32),
        pltpu.VMEM((1, 3, B), jnp.int32),
        pltpu.VMEM((B, 128), jnp.float32),
        pltpu.VMEM((B, 128), jnp.float32),
        pltpu.VMEM((B, 128), jnp.float32),
        pltpu.VMEM((1, 3, B), jnp.int32),
        pltpu.VMEM((B, 128), jnp.float32),
        pltpu.VMEM((4, D), jnp.float32),
        pltpu.VMEM((4, 16), jnp.float32),
        pltpu.SemaphoreType.DMA, pltpu.SemaphoreType.DMA,
        pltpu.SemaphoreType.DMA, pltpu.SemaphoreType.DMA,
        pltpu.SemaphoreType.DMA, pltpu.SemaphoreType.DMA,
        pltpu.SemaphoreType.DMA, pltpu.SemaphoreType.DMA,
    ],
)


EWP = E // NW


def _pred_body(row_hbm, col_hbm, u_hbm, v_hbm, out_hbm,
               rowv, colv, uv, vv, ob):
  c = lax.axis_index("c")
  s = lax.axis_index("s")
  wid = c * NS + s
  base = wid * EWP
  pltpu.sync_copy(
      (row_hbm.at[pl.ds(base, EWP)], col_hbm.at[pl.ds(base, EWP)],
       u_hbm, v_hbm),
      (rowv, colv, uv, vv))

  lanes = lax.iota(jnp.int32, 16)
  zero16 = jnp.zeros((16,), jnp.float32)

  def _go(i, _):
    r16 = rowv[pl.ds(i * 16, 16)]
    c16 = colv[pl.ds(i * 16, 16)]
    ov = zero16
    for j in range(16):
      uvec = uv[pl.ds(r16[j], 16)]
      vvec = vv[pl.ds(c16[j], 16)]
      ov = jnp.where(lanes == j, uvec[0] + vvec[0], ov)
    ob[pl.ds(i * 16, 16)] = ov
    return _
  lax.fori_loop(0, EWP // 16, _go, None)
  pltpu.sync_copy(ob, out_hbm.at[pl.ds(base, EWP)])


_pred_kernel = pl.kernel(
    _pred_body,
    out_type=jax.ShapeDtypeStruct((E,), jnp.float32),
    mesh=plsc.VectorSubcoreMesh(
        core_axis_name="c", subcore_axis_name="s",
        num_cores=NC, num_subcores=NS),
    scratch_types=[
        pltpu.VMEM((EWP,), jnp.int32),
        pltpu.VMEM((EWP,), jnp.int32),
        pltpu.VMEM((NP,), jnp.float32),
        pltpu.VMEM((NP,), jnp.float32),
        pltpu.VMEM((EWP,), jnp.float32),
    ],
)


def _blockdiag(att):
  a = att.reshape(H, C)
  return (a[:, :, None] * jnp.eye(H, dtype=jnp.float32)[:, None, :]
          ).reshape(D, H)


def kernel(x, edge_index, edge_attr, atom_emb, W, att_src, att_dst, att_edge,
           bias, edge_emb, bn_gamma, bn_beta, pred_W, pred_b):
  f32 = jnp.float32
  src = edge_index[0].astype(jnp.int32)
  dst = edge_index[1].astype(jnp.int32)
  attr = edge_attr.astype(jnp.int32)

  zpad = jnp.zeros((EP - E,), jnp.int32)
  srcp = jnp.concatenate([src, zpad])
  dstp = jnp.concatenate([dst, zpad])
  attrp = jnp.concatenate([attr, zpad])
  epk = jnp.stack([srcp, dstp, attrp], axis=0).reshape(3, NW * NB, B)
  epk = jnp.transpose(epk, (1, 0, 2))
  epk = jnp.concatenate([epk, jnp.zeros((4, 3, B), jnp.int32)], axis=0)

  xp = jnp.concatenate([x.astype(jnp.int32),
                        jnp.zeros((NP - N,), jnp.int32)])

  rex = jnp.kron(jnp.eye(H, dtype=f32), jnp.ones((1, C), f32))

  hp = _tc_embed(xp, atom_emb.astype(f32))

  for l in range(L):
    a2 = jnp.concatenate(
        [_blockdiag(att_src[l]), _blockdiag(att_dst[l])], axis=1)
    ea_l = edge_emb[l]
    aet = (ea_l.reshape(4, H, C) * att_edge[l].reshape(1, H, C)).sum(-1)
    aet16 = jnp.concatenate([aet, jnp.zeros((4, 8), f32)], axis=1)

    xs, asd, ads = _tc_pre(hp, W[l], a2)
    asd8 = asd.reshape(NP // 8, 128)
    ads8 = ads.reshape(NP // 8, 128)
    accm, accaux = _edge_kernel(epk, xs, asd8, ads8, ea_l, aet16)
    aux = accaux.reshape(NC, NP, 16)
    hp = _tc_post(accm[0], accm[1], aux[0], aux[1], xs, asd, hp,
                  ea_l, aet16, rex,
                  bias[l].reshape(1, D), bn_gamma[l].reshape(1, D),
                  bn_beta[l].reshape(1, D))

  pw16 = jnp.concatenate(
      [pred_W[:D], pred_W[D:], jnp.zeros((D, 14), f32)], axis=1)
  pb16 = jnp.zeros((1, 16), f32).at[0, 0].set(pred_b[0])
  uvt = _tc_uv(hp, pw16, pb16)
  u = uvt[:, 0]
  v = uvt[:, 1]

  pred = _pred_kernel(src, dst, u, v)
  return pred.reshape(E, 1)


# speedup vs baseline: 78.5784x; 1.8054x over previous
import functools

import jax
import jax.numpy as jnp
from jax import lax
from jax.experimental import pallas as pl
from jax.experimental.pallas import tpu as pltpu
from jax.experimental.pallas import tpu_sc as plsc

L = 3
D = 128
H = 8
C = 16
N = 10000
E = 320000

NC = 2
NS = 16
NW = NC * NS

NP = 10240
B = 32
NB = 316
EW = NB * B
EP = NW * EW
RPW = NP // NS
RPW8 = NP // 8 // NS

_BN_SCALE = 1.0 / (1.0 + 1e-5) ** 0.5


_RB = 512
_GRID = NP // _RB


def _embed_body(x_ref, emb_ref, o_ref):
  xv = x_ref[0, 0, :]
  oh = (xv[:, None] == lax.broadcasted_iota(jnp.int32, (1, 16), 1)
        ).astype(jnp.float32)
  o_ref[...] = jnp.dot(oh, emb_ref[...],
                       preferred_element_type=jnp.float32)


def _tc_embed(xp, atom_emb):
  xp3 = xp.reshape(_GRID, 1, _RB)
  return pl.pallas_call(
      _embed_body,
      grid=(_GRID,),
      in_specs=[
          pl.BlockSpec((1, 1, _RB), lambda i: (i, 0, 0)),
          pl.BlockSpec((16, D), lambda i: (0, 0)),
      ],
      out_specs=pl.BlockSpec((_RB, D), lambda i: (i, 0)),
      out_shape=jax.ShapeDtypeStruct((NP, D), jnp.float32),
  )(xp3, atom_emb)


def _pre_body(h_ref, w_ref, a2_ref, xs_ref, asd_ref, ads_ref):
  xs = jnp.dot(h_ref[...], w_ref[...], preferred_element_type=jnp.float32)
  asd = jnp.dot(xs, a2_ref[...], preferred_element_type=jnp.float32)
  xs_ref[...] = xs
  asd_ref[...] = asd
  ads_ref[...] = jnp.concatenate([asd[:, 8:], asd[:, :8]], axis=1)


def _tc_pre(hp, w_l, a2_l):
  return pl.pallas_call(
      _pre_body,
      grid=(_GRID,),
      in_specs=[
          pl.BlockSpec((_RB, D), lambda i: (i, 0)),
          pl.BlockSpec((D, D), lambda i: (0, 0)),
          pl.BlockSpec((D, 16), lambda i: (0, 0)),
      ],
      out_specs=[
          pl.BlockSpec((_RB, D), lambda i: (i, 0)),
          pl.BlockSpec((_RB, 16), lambda i: (i, 0)),
          pl.BlockSpec((_RB, 16), lambda i: (i, 0)),
      ],
      out_shape=[
          jax.ShapeDtypeStruct((NP, D), jnp.float32),
          jax.ShapeDtypeStruct((NP, 16), jnp.float32),
          jax.ShapeDtypeStruct((NP, 16), jnp.float32),
      ],
  )(hp, w_l, a2_l)


def _post_body(acc0_ref, acc1_ref, aux0_ref, aux1_ref, xs_ref, asd_ref,
               h_ref, ea_ref, aet_ref, rex_ref, bias_ref, gam_ref, bet_ref,
               o_ref):
  acc = acc0_ref[...] + acc1_ref[...]
  aux = aux0_ref[...] + aux1_ref[...]
  den_e = aux[:, :8]
  cc = aux[:, 8:12]
  cnt = jnp.sum(cc, axis=1)
  inv = 1.0 / jnp.maximum(cnt, 1.0)
  loop_attr = jnp.dot(cc, ea_ref[...],
                      preferred_element_type=jnp.float32) * inv[:, None]
  ael = jnp.dot(cc, aet_ref[...],
                preferred_element_type=jnp.float32)[:, :8] * inv[:, None]
  asd = asd_ref[...]
  a_src = asd[:, :8]
  a_dst = asd[:, 8:]
  al = a_dst + a_src + ael
  al = jnp.where(al > 0, al, 0.2 * al)
  exl = jnp.exp(al)
  rex = rex_ref[...]
  exl_e = jnp.dot(exl, rex, preferred_element_type=jnp.float32)
  den = jnp.dot(den_e + exl, rex, preferred_element_type=jnp.float32)
  acc_tot = acc + exl_e * (xs_ref[...] + loop_attr)
  out = acc_tot / den + bias_ref[...]
  out = out * (gam_ref[...] * _BN_SCALE) + bet_ref[...]
  o_ref[...] = jnp.maximum(out, 0.0) + h_ref[...]


def _tc_post(acc0, acc1, aux0, aux1, xs, asd, hp, ea_l, aet16_l, rex,
             bias_l, gam_l, bet_l):
  blk = lambda w: pl.BlockSpec((_RB, w), lambda i: (i, 0))
  full = lambda a, b: pl.BlockSpec((a, b), lambda i: (0, 0))
  return pl.pallas_call(
      _post_body,
      grid=(_GRID,),
      in_specs=[
          blk(D), blk(D), blk(16), blk(16), blk(D), blk(16), blk(D),
          full(4, D), full(4, 16), full(8, D),
          full(1, D), full(1, D), full(1, D),
      ],
      out_specs=blk(D),
      out_shape=jax.ShapeDtypeStruct((NP, D), jnp.float32),
  )(acc0, acc1, aux0, aux1, xs, asd, hp, ea_l, aet16_l, rex,
    bias_l, gam_l, bet_l)


def _uv_body(h_ref, pw_ref, pb_ref, o_ref):
  o_ref[...] = jnp.dot(h_ref[...], pw_ref[...],
                       preferred_element_type=jnp.float32) + pb_ref[...]


def _tc_uv(hp, pw16, pb16):
  return pl.pallas_call(
      _uv_body,
      grid=(_GRID,),
      in_specs=[
          pl.BlockSpec((_RB, D), lambda i: (i, 0)),
          pl.BlockSpec((D, 16), lambda i: (0, 0)),
          pl.BlockSpec((1, 16), lambda i: (0, 0)),
      ],
      out_specs=pl.BlockSpec((_RB, 16), lambda i: (i, 0)),
      out_shape=jax.ShapeDtypeStruct((NP, 16), jnp.float32),
  )(hp, pw16, pb16)


NBX = NW * NB + 4


def _edge_body(epk_hbm, xs_hbm, asd8_hbm, ads8_hbm,
               ea_hbm, aet_hbm, accm_out, accaux_out,
               accm_sh, accaux_sh,
               srcA, dstA, attrA, src8A, dst8A,
               srcB, dstB, attrB, src8B, dst8B,
               xbufA, sbufA, dbufA, epkA,
               xbufB, sbufB, dbufB, epkB,
               msgb, auxb, eav, aetv,
               gxA, gsA, gdA, giA, gxB, gsB, gdB, giB):
  c = lax.axis_index("c")
  s = lax.axis_index("s")
  wid = c * NS + s
  lanes = lax.iota(jnp.int32, 16)
  zero16 = jnp.zeros((16,), jnp.float32)
  bbase = wid * NB

  A = (srcA, dstA, attrA, src8A, dst8A, xbufA, sbufA, dbufA, epkA,
       gxA, gsA, gdA, giA)
  Bb = (srcB, dstB, attrB, src8B, dst8B, xbufB, sbufB, dbufB, epkB,
        gxB, gsB, gdB, giB)

  def _zrow(i, _):
    for k in range(8):
      xbufA[i, pl.ds(k * 16, 16)] = zero16
      auxb[i, pl.ds(k * 16, 16)] = zero16
    return _
  lax.fori_loop(0, B, _zrow, None)

  def _zcp(j, _):
    pltpu.sync_copy(xbufA, accm_sh.at[pl.ds(s * RPW + j * B, B)])
    return _
  lax.fori_loop(0, RPW // B, _zcp, None)

  def _zcpa(j, _):
    pltpu.sync_copy(auxb.at[pl.ds(0, 16)],
                    accaux_sh.at[pl.ds(s * RPW8 + j * 16, 16)])
    return _
  lax.fori_loop(0, RPW8 // 16, _zcpa, None)

  pltpu.sync_copy((ea_hbm, aet_hbm), (eav, aetv))
  plsc.subcore_barrier()

  def i_desc(blk, bufs):
    epk, gi = bufs[8], bufs[12]
    return pltpu.make_async_copy(epk_hbm.at[pl.ds(bbase + blk, 1)], epk, gi)

  def prep(bufs):
    srcv, dstv, attrv, srcv8, dstv8, epk = bufs[:5] + (bufs[8],)
    for g in range(B // 16):
      o = g * 16
      sg = epk[0, 0, pl.ds(o, 16)]
      dg = epk[0, 1, pl.ds(o, 16)]
      srcv[pl.ds(o, 16)] = sg
      dstv[pl.ds(o, 16)] = dg
      attrv[pl.ds(o, 16)] = epk[0, 2, pl.ds(o, 16)]
      srcv8[pl.ds(o, 16)] = lax.shift_right_logical(sg, 3)
      dstv8[pl.ds(o, 16)] = lax.shift_right_logical(dg, 3)

  def g_descs(bufs):
    srcv8, dstv8 = bufs[3], bufs[4]
    srcv, xbuf, sbuf, dbuf = bufs[0], bufs[5], bufs[6], bufs[7]
    gx, gs, gd = bufs[9:12]
    return (pltpu.make_async_copy(xs_hbm.at[srcv], xbuf, gx),
            pltpu.make_async_copy(asd8_hbm.at[srcv8], sbuf, gs),
            pltpu.make_async_copy(ads8_hbm.at[dstv8], dbuf, gd))

  def g_start(bufs):
    for d in g_descs(bufs):
      d.start()

  def g_wait(bufs):
    for d in g_descs(bufs):
      d.wait()

  def compute(bufs):
    srcv, dstv, attrv = bufs[:3]
    xbuf, sbuf, dbuf = bufs[5:8]

    @functools.partial(plsc.parallel_loop, 0, B // 16)
    def _group(g):
      srcg = srcv[pl.ds(g * 16, 16)]
      dstg = dstv[pl.ds(g * 16, 16)]
      attrg = attrv[pl.ds(g * 16, 16)]
      for j in range(16):
        e = g * 16 + j
        src_j = srcg[j]
        dst_j = dstg[j]
        a_e = attrg[j]
        maskf = jnp.where(src_j != dst_j, 1.0, 0.0)
        so8 = (src_j & 7) * 16
        do8 = (dst_j & 7) * 16
        sv = (sbuf[e, pl.ds(so8, 16)] + dbuf[e, pl.ds(do8, 16)]
              + aetv[a_e, pl.ds(0, 16)])
        tv = jnp.where(sv > 0, sv, 0.2 * sv)
        exv = jnp.exp(tv) * maskf
        oh = jnp.where(lanes == 8 + a_e, maskf, 0.0)
        aux16 = jnp.where(lanes < 8, exv, oh)
        auxb[e, pl.ds(do8, 16)] = aux16
        for h in range(H):
          bex = jnp.full((16,), exv[h])
          xv = xbuf[e, pl.ds(h * 16, 16)]
          ev = eav[a_e, pl.ds(h * 16, 16)]
          msgb[e, pl.ds(h * 16, 16)] = bex * (xv + ev)

  def scatter_rezero(bufs):
    dstv, dstv8 = bufs[1], bufs[4]
    pltpu.sync_copy((msgb, auxb), (accm_sh.at[dstv], accaux_sh.at[dstv8]),
                    add=True)

    def _z(g, _):
      dstg = dstv[pl.ds(g * 16, 16)]
      for j in range(16):
        auxb[g * 16 + j, pl.ds((dstg[j] & 7) * 16, 16)] = zero16
      return _
    lax.fori_loop(0, B // 16, _z, None)

  i_desc(0, A).start()
  i_desc(1, Bb).start()
  i_desc(0, A).wait()
  prep(A)
  g_start(A)
  i_desc(1, Bb).wait()
  prep(Bb)
  g_start(Bb)
  i_desc(2, A).start()
  i_desc(3, Bb).start()

  def _step(blk, bufs):
    g_wait(bufs)
    compute(bufs)
    scatter_rezero(bufs)
    i_desc(blk + 2, bufs).wait()
    prep(bufs)
    g_start(bufs)
    i_desc(blk + 4, bufs).start()

  def _iter(i, _):
    _step(2 * i, A)
    _step(2 * i + 1, Bb)
    return _
  lax.fori_loop(0, NB // 2, _iter, None)

  g_wait(A)
  g_wait(Bb)
  i_desc(NB + 2, A).wait()
  i_desc(NB + 3, Bb).wait()

  plsc.subcore_barrier()

  def _cpo(j, _):
    r = s * RPW + j * B
    pltpu.sync_copy(accm_sh.at[pl.ds(r, B)], xbufA)
    pltpu.sync_copy(xbufA, accm_out.at[c, pl.ds(r, B)])
    return _
  lax.fori_loop(0, RPW // B, _cpo, None)

  def _cpoa(j, _):
    r = s * RPW8 + j * 16
    pltpu.sync_copy(accaux_sh.at[pl.ds(r, 16)], dbufA.at[pl.ds(0, 16)])
    pltpu.sync_copy(dbufA.at[pl.ds(0, 16)], accaux_out.at[c, pl.ds(r, 16)])
    return _
  lax.fori_loop(0, RPW8 // 16, _cpoa, None)


def _i32buf():
  return pltpu.VMEM((B,), jnp.int32)


_edge_kernel = pl.kernel(
    _edge_body,
    out_type=[
        jax.ShapeDtypeStruct((NC, NP, D), jnp.float32),
        jax.ShapeDtypeStruct((NC, NP // 8, 128), jnp.float32),
    ],
    mesh=plsc.VectorSubcoreMesh(
        core_axis_name="c", subcore_axis_name="s",
        num_cores=NC, num_subcores=NS),
    scratch_types=[
        pltpu.VMEM_SHARED((NP, D), jnp.float32),
        pltpu.VMEM_SHARED((NP // 8, 128), jnp.float32),
        _i32buf(), _i32buf(), _i32buf(), _i32buf(), _i32buf(),
        _i32buf(), _i32buf(), _i32buf(), _i32buf(), _i32buf(),
        pltpu.VMEM((B, 128), jnp.float32),
        pltpu.VMEM((B, 128), jnp.float32),
        pltpu.VMEM((B, 128), jnp.float32),
        pltpu.VMEM((1, 3, B), jnp.int32),
        pltpu.VMEM((B, 128), jnp.float32),
        pltpu.VMEM((B, 128), jnp.float32),
        pltpu.VMEM((B, 128), jnp.float32),
        pltpu.VMEM((1, 3, B), jnp.int32),
        pltpu.VMEM((B, 128), jnp.float32),
        pltpu.VMEM((B, 128), jnp.float32),
        pltpu.VMEM((4, D), jnp.float32),
        pltpu.VMEM((4, 16), jnp.float32),
        pltpu.SemaphoreType.DMA, pltpu.SemaphoreType.DMA,
        pltpu.SemaphoreType.DMA, pltpu.SemaphoreType.DMA,
        pltpu.SemaphoreType.DMA, pltpu.SemaphoreType.DMA,
        pltpu.SemaphoreType.DMA, pltpu.SemaphoreType.DMA,
    ],
)


EWP = E // NW


def _pred_body(row_hbm, col_hbm, u_hbm, v_hbm, out_hbm,
               rowv, colv, uv, vv, ob):
  c = lax.axis_index("c")
  s = lax.axis_index("s")
  wid = c * NS + s
  base = wid * EWP
  pltpu.sync_copy(
      (row_hbm.at[pl.ds(base, EWP)], col_hbm.at[pl.ds(base, EWP)],
       u_hbm, v_hbm),
      (rowv, colv, uv, vv))

  lanes = lax.iota(jnp.int32, 16)
  zero16 = jnp.zeros((16,), jnp.float32)

  def _go(i, _):
    r16 = rowv[pl.ds(i * 16, 16)]
    c16 = colv[pl.ds(i * 16, 16)]
    ov = zero16
    for j in range(16):
      uvec = uv[pl.ds(r16[j], 16)]
      vvec = vv[pl.ds(c16[j], 16)]
      ov = jnp.where(lanes == j, uvec[0] + vvec[0], ov)
    ob[pl.ds(i * 16, 16)] = ov
    return _
  lax.fori_loop(0, EWP // 16, _go, None)
  pltpu.sync_copy(ob, out_hbm.at[pl.ds(base, EWP)])


_pred_kernel = pl.kernel(
    _pred_body,
    out_type=jax.ShapeDtypeStruct((E,), jnp.float32),
    mesh=plsc.VectorSubcoreMesh(
        core_axis_name="c", subcore_axis_name="s",
        num_cores=NC, num_subcores=NS),
    scratch_types=[
        pltpu.VMEM((EWP,), jnp.int32),
        pltpu.VMEM((EWP,), jnp.int32),
        pltpu.VMEM((NP,), jnp.float32),
        pltpu.VMEM((NP,), jnp.float32),
        pltpu.VMEM((EWP,), jnp.float32),
    ],
)


def _blockdiag(att):
  a = att.reshape(H, C)
  return (a[:, :, None] * jnp.eye(H, dtype=jnp.float32)[:, None, :]
          ).reshape(D, H)


def kernel(x, edge_index, edge_attr, atom_emb, W, att_src, att_dst, att_edge,
           bias, edge_emb, bn_gamma, bn_beta, pred_W, pred_b):
  f32 = jnp.float32
  src = edge_index[0].astype(jnp.int32)
  dst = edge_index[1].astype(jnp.int32)
  attr = edge_attr.astype(jnp.int32)

  zpad = jnp.zeros((EP - E,), jnp.int32)
  srcp = jnp.concatenate([src, zpad])
  dstp = jnp.concatenate([dst, zpad])
  attrp = jnp.concatenate([attr, zpad])
  epk = jnp.stack([srcp, dstp, attrp], axis=0).reshape(3, NW * NB, B)
  epk = jnp.transpose(epk, (1, 0, 2))
  epk = jnp.concatenate([epk, jnp.zeros((4, 3, B), jnp.int32)], axis=0)

  xp = jnp.concatenate([x.astype(jnp.int32),
                        jnp.zeros((NP - N,), jnp.int32)])

  rex = jnp.kron(jnp.eye(H, dtype=f32), jnp.ones((1, C), f32))

  hp = _tc_embed(xp, atom_emb.astype(f32))

  for l in range(L):
    a2 = jnp.concatenate(
        [_blockdiag(att_src[l]), _blockdiag(att_dst[l])], axis=1)
    ea_l = edge_emb[l]
    aet = (ea_l.reshape(4, H, C) * att_edge[l].reshape(1, H, C)).sum(-1)
    aet16 = jnp.concatenate([aet, jnp.zeros((4, 8), f32)], axis=1)

    xs, asd, ads = _tc_pre(hp, W[l], a2)
    asd8 = asd.reshape(NP // 8, 128)
    ads8 = ads.reshape(NP // 8, 128)
    accm, accaux = _edge_kernel(epk, xs, asd8, ads8, ea_l, aet16)
    aux = accaux.reshape(NC, NP, 16)
    hp = _tc_post(accm[0], accm[1], aux[0], aux[1], xs, asd, hp,
                  ea_l, aet16, rex,
                  bias[l].reshape(1, D), bn_gamma[l].reshape(1, D),
                  bn_beta[l].reshape(1, D))

  pw16 = jnp.concatenate(
      [pred_W[:D], pred_W[D:], jnp.zeros((D, 14), f32)], axis=1)
  pb16 = jnp.zeros((1, 16), f32).at[0, 0].set(pred_b[0])
  uvt = _tc_uv(hp, pw16, pb16)
  u = uvt[:, 0]
  v = uvt[:, 1]

  pred = _pred_kernel(src, dst, u, v)
  return pred.reshape(E, 1)
